# baseline (speedup 1.0000x reference)
import jax
import jax.numpy as jnp
from jax.experimental import pallas as pl
from jax.experimental.pallas import tpu as pltpu

_B = 4096
_D = 1024
_NCPAD = 128
_TEMP = 0.1
_K = 5
_KG = 3
_BI = 256
_GRID = _B // _BI
_LANES = 128
_NTILES = _B // _LANES


def _prep_body(x_ref, y_ref, lab_ref, labcol_ref, xsc_ref, ysc_ref, w_ref,
               v_ref):
    xx = x_ref[...].astype(jnp.float32)
    xn2 = jnp.sum(xx * xx, axis=1, keepdims=True)
    xinv = 1.0 / jnp.maximum(jnp.sqrt(xn2), 1e-12)
    xsc_ref[...] = (xx * xinv).astype(jnp.bfloat16)
    yy = y_ref[...].astype(jnp.float32)
    n2 = jnp.sum(yy * yy, axis=1, keepdims=True)
    inv = (1.0 / _TEMP) / jnp.maximum(jnp.sqrt(n2), 1e-12)
    ysc_ref[...] = (yy * inv).astype(jnp.bfloat16)
    lab = lab_ref[...]
    cls = jax.lax.broadcasted_iota(jnp.int32, (_NCPAD, _B), 0)
    onehot = (lab == cls).astype(jnp.float32)
    counts = jnp.sum(onehot, axis=1, keepdims=True)
    invc = 1.0 / jnp.maximum(counts, 1.0)
    w_ref[...] = jnp.sum(onehot * invc, axis=0, keepdims=True)
    cls2 = jax.lax.broadcasted_iota(jnp.int32, (_B, _NCPAD), 1)
    v_ref[...] = (labcol_ref[...] == cls2).astype(jnp.bfloat16)


def _main_body(x_ref, ysc_ref, w_ref, lab_ref, labrow_ref, v_ref, out_ref,
               acc_ref):
    i = pl.program_id(0)

    @pl.when(i == 0)
    def _init():
        acc_ref[...] = jnp.zeros((1, 1), jnp.float32)

    sim = jax.lax.dot_general(
        x_ref[...], ysc_ref[...], (((1,), (1,)), ((), ())),
        preferred_element_type=jnp.float32)

    pos = labrow_ref[...] == lab_ref[...]
    wn = jnp.where(pos, 0.0, sim) * w_ref[...]

    esim = jnp.exp(sim)
    esim_bf = esim.astype(jnp.bfloat16)
    cls_sums = jax.lax.dot_general(
        esim_bf, v_ref[...], (((1,), (0,)), ((), ())),
        preferred_element_type=jnp.float32)
    ucls = jax.lax.broadcasted_iota(jnp.int32, (_BI, _NCPAD), 1)
    uown = (labrow_ref[...] == ucls).astype(jnp.float32)
    pos_sum = jnp.sum(cls_sums * uown, axis=1, keepdims=True)

    neg_inf = jnp.float32(-jnp.inf)
    r = [jnp.full((_BI, _LANES), neg_inf, jnp.float32) for _ in range(_KG)]
    for t in range(_NTILES):
        v = wn[:, t * _LANES:(t + 1) * _LANES]
        for j in range(_KG):
            hi = jnp.maximum(r[j], v)
            v = jnp.minimum(r[j], v)
            r[j] = hi
    planes = r
    thr = None
    for it in range(_K):
        mx = planes[0]
        for j in range(1, _KG):
            mx = jnp.maximum(mx, planes[j])
        mm = jnp.max(mx, axis=1, keepdims=True)
        if it == _K - 1:
            thr = mm
        else:
            planes = [jnp.where(p == mm, neg_inf, p) for p in planes]

    neg_sum = jnp.sum(jnp.where(wn >= thr, esim, 0.0), axis=1, keepdims=True)

    loss = -jnp.log(pos_sum / (pos_sum + neg_sum + 1e-8))
    acc_ref[...] += jnp.sum(loss, axis=0, keepdims=True)

    @pl.when(i == _GRID - 1)
    def _fin():
        out_ref[...] = acc_ref[...] / (jnp.float32(_B) + 1e-8)


def kernel(x, y, labels):
    lab2d = labels.reshape(1, _B).astype(jnp.int32)
    labcol = labels.reshape(_B, 1).astype(jnp.int32)

    xsc, ysc, w, v = pl.pallas_call(
        _prep_body,
        grid=(1,),
        in_specs=[
            pl.BlockSpec((_B, _D), lambda i: (0, 0)),
            pl.BlockSpec((_B, _D), lambda i: (0, 0)),
            pl.BlockSpec((1, _B), lambda i: (0, 0)),
            pl.BlockSpec((_B, 1), lambda i: (0, 0)),
        ],
        out_specs=[
            pl.BlockSpec((_B, _D), lambda i: (0, 0)),
            pl.BlockSpec((_B, _D), lambda i: (0, 0)),
            pl.BlockSpec((1, _B), lambda i: (0, 0)),
            pl.BlockSpec((_B, _NCPAD), lambda i: (0, 0)),
        ],
        out_shape=[
            jax.ShapeDtypeStruct((_B, _D), jnp.bfloat16),
            jax.ShapeDtypeStruct((_B, _D), jnp.bfloat16),
            jax.ShapeDtypeStruct((1, _B), jnp.float32),
            jax.ShapeDtypeStruct((_B, _NCPAD), jnp.bfloat16),
        ],
    )(x.astype(jnp.bfloat16), y.astype(jnp.bfloat16), lab2d, labcol)

    out = pl.pallas_call(
        _main_body,
        grid=(_GRID,),
        in_specs=[
            pl.BlockSpec((_BI, _D), lambda i: (i, 0)),
            pl.BlockSpec((_B, _D), lambda i: (0, 0)),
            pl.BlockSpec((1, _B), lambda i: (0, 0)),
            pl.BlockSpec((1, _B), lambda i: (0, 0)),
            pl.BlockSpec((_BI, 1), lambda i: (i, 0)),
            pl.BlockSpec((_B, _NCPAD), lambda i: (0, 0)),
        ],
        out_specs=pl.BlockSpec((1, 1), lambda i: (0, 0)),
        out_shape=jax.ShapeDtypeStruct((1, 1), jnp.float32),
        scratch_shapes=[pltpu.VMEM((1, 1), jnp.float32)],
    )(xsc, ysc, w, lab2d, labcol, v)
    return out.reshape(())


# speedup vs baseline: 1.0031x; 1.0031x over previous
import functools

import jax
import jax.numpy as jnp
from jax.experimental import pallas as pl
from jax.experimental.pallas import tpu as pltpu
from jax.experimental.pallas import tpu_sc as plsc

_B = 4096
_D = 1024
_NCPAD = 128
_TEMP = 0.1
_K = 5
_KG = 3
_BI = 256
_GRID = _B // _BI
_LANES = 128
_NTILES = _B // _LANES


_NC = 2
_NS = 16
_NW = _NC * _NS
_LPW = _B // _NW
_CPW = 4
_NVEC = _B // 16


def _cnt_sc_body(lab_hbm, cnt_hbm, lab_v, row_v):
    cid = jax.lax.axis_index("c")
    sid = jax.lax.axis_index("s")
    wid = sid * _NC + cid
    pltpu.sync_copy(lab_hbm, lab_v)
    c0 = wid * _CPW

    def body(k, accs):
        v = lab_v[pl.ds(k * 16, 16)]
        return tuple(accs[t] + jnp.where(v == c0 + t, 1, 0)
                     for t in range(_CPW))

    accs = jax.lax.fori_loop(
        0, _NVEC, body,
        tuple(jnp.zeros((16,), jnp.int32) for _ in range(_CPW)))
    for t in range(_CPW):
        row_v[pl.ds(t * 16, 16)] = accs[t].astype(jnp.float32)
    pltpu.sync_copy(row_v, cnt_hbm.at[pl.ds(wid * _CPW * 16, _CPW * 16)])


def _sc_counts(labels_i32):
    mesh = plsc.VectorSubcoreMesh(core_axis_name="c", subcore_axis_name="s")
    return pl.kernel(
        _cnt_sc_body,
        out_type=jax.ShapeDtypeStruct((_NW * _CPW * 16,), jnp.float32),
        mesh=mesh,
        scratch_types=[
            pltpu.VMEM((_B,), jnp.int32),
            pltpu.VMEM((_CPW * 16,), jnp.float32),
        ],
    )(labels_i32)


def _prep_body(y_ref, labcol_ref, lab_ref, cnt_ref, ysc_ref, v_ref, w_ref):
    yy = y_ref[...]
    n2 = jnp.sum(yy * yy, axis=1, keepdims=True)
    inv = (1.0 / _TEMP) / jnp.maximum(jnp.sqrt(n2), 1e-12)
    ysc_ref[...] = (yy * inv).astype(jnp.bfloat16)
    cls2 = jax.lax.broadcasted_iota(jnp.int32, (_B, _NCPAD), 1)
    v_ref[...] = (labcol_ref[...] == cls2).astype(jnp.bfloat16)
    lab = lab_ref[...]
    counts = jnp.sum(cnt_ref[...], axis=1, keepdims=True)
    invc = 1.0 / jnp.maximum(counts, 1.0)
    cls3 = jax.lax.broadcasted_iota(jnp.int32, (_NCPAD, _B), 0)
    w_ref[...] = jnp.sum(jnp.where(lab == cls3, invc, 0.0),
                         axis=0, keepdims=True)


def _main_body(x_ref, ysc_ref, w_ref, lab_ref, labrow_ref, v_ref, out_ref,
               acc_ref):
    i = pl.program_id(0)

    @pl.when(i == 0)
    def _init():
        acc_ref[...] = jnp.zeros((1, 1), jnp.float32)

    xb = x_ref[...]
    xn2 = jnp.sum(xb * xb, axis=1, keepdims=True)
    rowinv = 1.0 / jnp.maximum(jnp.sqrt(xn2), 1e-12)
    xs = (xb * rowinv).astype(jnp.bfloat16)
    sim = jax.lax.dot_general(
        xs, ysc_ref[...], (((1,), (1,)), ((), ())),
        preferred_element_type=jnp.float32)

    pos = labrow_ref[...] == lab_ref[...]
    wn = jnp.where(pos, 0.0, sim) * w_ref[...]

    esim = jnp.exp(sim)
    esim_bf = esim.astype(jnp.bfloat16)
    cls_sums = jax.lax.dot_general(
        esim_bf, v_ref[...], (((1,), (0,)), ((), ())),
        preferred_element_type=jnp.float32)
    ucls = jax.lax.broadcasted_iota(jnp.int32, (_BI, _NCPAD), 1)
    uown = (labrow_ref[...] == ucls).astype(jnp.float32)
    pos_sum = jnp.sum(cls_sums * uown, axis=1, keepdims=True)

    neg_inf = jnp.float32(-jnp.inf)
    r = [jnp.full((_BI, _LANES), neg_inf, jnp.float32) for _ in range(_KG)]
    for t in range(_NTILES):
        v = wn[:, t * _LANES:(t + 1) * _LANES]
        for j in range(_KG):
            hi = jnp.maximum(r[j], v)
            v = jnp.minimum(r[j], v)
            r[j] = hi
    planes = r
    thr = None
    for it in range(_K):
        mx = planes[0]
        for j in range(1, _KG):
            mx = jnp.maximum(mx, planes[j])
        mm = jnp.max(mx, axis=1, keepdims=True)
        if it == _K - 1:
            thr = mm
        else:
            planes = [jnp.where(p == mm, neg_inf, p) for p in planes]

    neg_sum = jnp.sum(jnp.where(wn >= thr, esim, 0.0), axis=1, keepdims=True)

    loss = -jnp.log(pos_sum / (pos_sum + neg_sum + 1e-8))
    acc_ref[...] += jnp.sum(loss, axis=0, keepdims=True)

    @pl.when(i == _GRID - 1)
    def _fin():
        out_ref[...] = acc_ref[...] / (jnp.float32(_B) + 1e-8)


def kernel(x, y, labels):
    lab2d = labels.reshape(1, _B).astype(jnp.int32)
    labcol = labels.reshape(_B, 1).astype(jnp.int32)

    counts = _sc_counts(labels.astype(jnp.int32)).reshape(_NCPAD, 16)
    ysc, v, w = pl.pallas_call(
        _prep_body,
        grid=(1,),
        in_specs=[
            pl.BlockSpec((_B, _D), lambda i: (0, 0)),
            pl.BlockSpec((_B, 1), lambda i: (0, 0)),
            pl.BlockSpec((1, _B), lambda i: (0, 0)),
            pl.BlockSpec((_NCPAD, 16), lambda i: (0, 0)),
        ],
        out_specs=[
            pl.BlockSpec((_B, _D), lambda i: (0, 0)),
            pl.BlockSpec((_B, _NCPAD), lambda i: (0, 0)),
            pl.BlockSpec((1, _B), lambda i: (0, 0)),
        ],
        out_shape=[
            jax.ShapeDtypeStruct((_B, _D), jnp.bfloat16),
            jax.ShapeDtypeStruct((_B, _NCPAD), jnp.bfloat16),
            jax.ShapeDtypeStruct((1, _B), jnp.float32),
        ],
    )(y, labcol, lab2d, counts)

    out = pl.pallas_call(
        _main_body,
        grid=(_GRID,),
        in_specs=[
            pl.BlockSpec((_BI, _D), lambda i: (i, 0)),
            pl.BlockSpec((_B, _D), lambda i: (0, 0)),
            pl.BlockSpec((1, _B), lambda i: (0, 0)),
            pl.BlockSpec((1, _B), lambda i: (0, 0)),
            pl.BlockSpec((_BI, 1), lambda i: (i, 0)),
            pl.BlockSpec((_B, _NCPAD), lambda i: (0, 0)),
        ],
        out_specs=pl.BlockSpec((1, 1), lambda i: (0, 0)),
        out_shape=jax.ShapeDtypeStruct((1, 1), jnp.float32),
        scratch_shapes=[pltpu.VMEM((1, 1), jnp.float32)],
    )(x, ysc, w, lab2d, labcol, v)
    return out.reshape(())


# speedup vs baseline: 1.0310x; 1.0278x over previous
import functools

import jax
import jax.numpy as jnp
from jax.experimental import pallas as pl
from jax.experimental.pallas import tpu as pltpu
from jax.experimental.pallas import tpu_sc as plsc

_B = 4096
_D = 1024
_NCPAD = 128
_TEMP = 0.1
_K = 5
_KG = 3
_BI = 256
_GRID = _B // _BI
_LANES = 128
_NTILES = _B // _LANES


_NC = 2
_NS = 16
_NW = _NC * _NS
_LPW = _B // _NW
_CPW = 4
_NVEC = _B // 16


def _cnt_sc_body(lab_hbm, cnt_hbm, lab_v, row_v):
    cid = jax.lax.axis_index("c")
    sid = jax.lax.axis_index("s")
    wid = sid * _NC + cid
    pltpu.sync_copy(lab_hbm, lab_v)
    c0 = wid * _CPW

    def body(k, accs):
        v = lab_v[pl.ds(k * 16, 16)]
        return tuple(accs[t] + jnp.where(v == c0 + t, 1, 0)
                     for t in range(_CPW))

    accs = jax.lax.fori_loop(
        0, _NVEC, body,
        tuple(jnp.zeros((16,), jnp.int32) for _ in range(_CPW)))
    for t in range(_CPW):
        row_v[pl.ds(t * 16, 16)] = accs[t].astype(jnp.float32)
    pltpu.sync_copy(row_v, cnt_hbm.at[pl.ds(wid * _CPW * 16, _CPW * 16)])


def _sc_counts(labels_i32):
    mesh = plsc.VectorSubcoreMesh(core_axis_name="c", subcore_axis_name="s")
    return pl.kernel(
        _cnt_sc_body,
        out_type=jax.ShapeDtypeStruct((_NW * _CPW * 16,), jnp.float32),
        mesh=mesh,
        scratch_types=[
            pltpu.VMEM((_B,), jnp.int32),
            pltpu.VMEM((_CPW * 16,), jnp.float32),
        ],
    )(labels_i32)


def _prep_body(y_ref, labcol_ref, ysc_ref, v_ref):
    yy = y_ref[...]
    n2 = jnp.sum(yy * yy, axis=1, keepdims=True)
    inv = (1.0 / _TEMP) / jnp.maximum(jnp.sqrt(n2), 1e-12)
    ysc_ref[...] = (yy * inv).astype(jnp.bfloat16)
    cls2 = jax.lax.broadcasted_iota(jnp.int32, (_B, _NCPAD), 1)
    v_ref[...] = (labcol_ref[...] == cls2).astype(jnp.bfloat16)


def _main_body(x_ref, ysc_ref, cnt_ref, lab_ref, labrow_ref, v_ref, out_ref,
               acc_ref, w_ref):
    i = pl.program_id(0)

    @pl.when(i == 0)
    def _init():
        acc_ref[...] = jnp.zeros((1, 1), jnp.float32)
        counts = jnp.sum(cnt_ref[...], axis=1, keepdims=True)
        invc = 1.0 / jnp.maximum(counts, 1.0)
        cls3 = jax.lax.broadcasted_iota(jnp.int32, (_NCPAD, _B), 0)
        w_ref[...] = jnp.sum(jnp.where(lab_ref[...] == cls3, invc, 0.0),
                             axis=0, keepdims=True)

    xb = x_ref[...]
    xn2 = jnp.sum(xb * xb, axis=1, keepdims=True)
    rowinv = 1.0 / jnp.maximum(jnp.sqrt(xn2), 1e-12)
    xs = (xb * rowinv).astype(jnp.bfloat16)
    sim = jax.lax.dot_general(
        xs, ysc_ref[...], (((1,), (1,)), ((), ())),
        preferred_element_type=jnp.float32)

    pos = labrow_ref[...] == lab_ref[...]
    wn = jnp.where(pos, 0.0, sim) * w_ref[...]

    esim = jnp.exp(sim)
    esim_bf = esim.astype(jnp.bfloat16)
    cls_sums = jax.lax.dot_general(
        esim_bf, v_ref[...], (((1,), (0,)), ((), ())),
        preferred_element_type=jnp.float32)
    ucls = jax.lax.broadcasted_iota(jnp.int32, (_BI, _NCPAD), 1)
    uown = (labrow_ref[...] == ucls).astype(jnp.float32)
    pos_sum = jnp.sum(cls_sums * uown, axis=1, keepdims=True)

    neg_inf = jnp.float32(-jnp.inf)
    r = [jnp.full((_BI, _LANES), neg_inf, jnp.float32) for _ in range(_KG)]
    for t in range(_NTILES):
        v = wn[:, t * _LANES:(t + 1) * _LANES]
        for j in range(_KG):
            hi = jnp.maximum(r[j], v)
            v = jnp.minimum(r[j], v)
            r[j] = hi
    planes = r
    thr = None
    for it in range(_K):
        mx = planes[0]
        for j in range(1, _KG):
            mx = jnp.maximum(mx, planes[j])
        mm = jnp.max(mx, axis=1, keepdims=True)
        if it == _K - 1:
            thr = mm
        else:
            planes = [jnp.where(p == mm, neg_inf, p) for p in planes]

    neg_sum = jnp.sum(jnp.where(wn >= thr, esim, 0.0), axis=1, keepdims=True)

    loss = -jnp.log(pos_sum / (pos_sum + neg_sum + 1e-8))
    acc_ref[...] += jnp.sum(loss, axis=0, keepdims=True)

    @pl.when(i == _GRID - 1)
    def _fin():
        out_ref[...] = acc_ref[...] / (jnp.float32(_B) + 1e-8)


def kernel(x, y, labels):
    lab2d = labels.reshape(1, _B).astype(jnp.int32)
    labcol = labels.reshape(_B, 1).astype(jnp.int32)

    counts = _sc_counts(labels.astype(jnp.int32)).reshape(_NCPAD, 16)
    ysc, v = pl.pallas_call(
        _prep_body,
        grid=(1,),
        in_specs=[
            pl.BlockSpec((_B, _D), lambda i: (0, 0)),
            pl.BlockSpec((_B, 1), lambda i: (0, 0)),
        ],
        out_specs=[
            pl.BlockSpec((_B, _D), lambda i: (0, 0)),
            pl.BlockSpec((_B, _NCPAD), lambda i: (0, 0)),
        ],
        out_shape=[
            jax.ShapeDtypeStruct((_B, _D), jnp.bfloat16),
            jax.ShapeDtypeStruct((_B, _NCPAD), jnp.bfloat16),
        ],
    )(y, labcol)

    out = pl.pallas_call(
        _main_body,
        grid=(_GRID,),
        in_specs=[
            pl.BlockSpec((_BI, _D), lambda i: (i, 0)),
            pl.BlockSpec((_B, _D), lambda i: (0, 0)),
            pl.BlockSpec((_NCPAD, 16), lambda i: (0, 0)),
            pl.BlockSpec((1, _B), lambda i: (0, 0)),
            pl.BlockSpec((_BI, 1), lambda i: (i, 0)),
            pl.BlockSpec((_B, _NCPAD), lambda i: (0, 0)),
        ],
        out_specs=pl.BlockSpec((1, 1), lambda i: (0, 0)),
        out_shape=jax.ShapeDtypeStruct((1, 1), jnp.float32),
        scratch_shapes=[pltpu.VMEM((1, 1), jnp.float32),
                        pltpu.VMEM((1, _B), jnp.float32)],
    )(x, ysc, counts, lab2d, labcol, v)
    return out.reshape(())


# speedup vs baseline: 1.0905x; 1.0577x over previous
import functools

import jax
import jax.numpy as jnp
from jax.experimental import pallas as pl
from jax.experimental.pallas import tpu as pltpu
from jax.experimental.pallas import tpu_sc as plsc

_B = 4096
_D = 1024
_NCPAD = 128
_TEMP = 0.1
_K = 5
_KG = 2
_BI = 256
_GRID = _B // _BI
_LANES = 128
_NTILES = _B // _LANES


_NC = 2
_NS = 16
_NW = _NC * _NS
_LPW = _B // _NW
_CPW = 4
_NVEC = _B // 16


def _cnt_sc_body(lab_hbm, cnt_hbm, lab_v, row_v):
    cid = jax.lax.axis_index("c")
    sid = jax.lax.axis_index("s")
    wid = sid * _NC + cid
    pltpu.sync_copy(lab_hbm, lab_v)
    c0 = wid * _CPW

    def body(k, accs):
        v = lab_v[pl.ds(k * 16, 16)]
        return tuple(accs[t] + jnp.where(v == c0 + t, 1, 0)
                     for t in range(_CPW))

    accs = jax.lax.fori_loop(
        0, _NVEC, body,
        tuple(jnp.zeros((16,), jnp.int32) for _ in range(_CPW)))
    for t in range(_CPW):
        row_v[pl.ds(t * 16, 16)] = accs[t].astype(jnp.float32)
    pltpu.sync_copy(row_v, cnt_hbm.at[pl.ds(wid * _CPW * 16, _CPW * 16)])


def _sc_counts(labels_i32):
    mesh = plsc.VectorSubcoreMesh(core_axis_name="c", subcore_axis_name="s")
    return pl.kernel(
        _cnt_sc_body,
        out_type=jax.ShapeDtypeStruct((_NW * _CPW * 16,), jnp.float32),
        mesh=mesh,
        scratch_types=[
            pltpu.VMEM((_B,), jnp.int32),
            pltpu.VMEM((_CPW * 16,), jnp.float32),
        ],
    )(labels_i32)


def _prep_body(y_ref, labcol_ref, ysc_ref, v_ref):
    yy = y_ref[...]
    n2 = jnp.sum(yy * yy, axis=1, keepdims=True)
    inv = (1.0 / _TEMP) / jnp.maximum(jnp.sqrt(n2), 1e-12)
    ysc_ref[...] = (yy * inv).astype(jnp.bfloat16)
    cls2 = jax.lax.broadcasted_iota(jnp.int32, (_B, _NCPAD), 1)
    v_ref[...] = (labcol_ref[...] == cls2).astype(jnp.bfloat16)


def _main_body(x_ref, ysc_ref, cnt_ref, lab_ref, labrow_ref, v_ref, out_ref,
               acc_ref, w_ref):
    i = pl.program_id(0)

    @pl.when(i == 0)
    def _init():
        acc_ref[...] = jnp.zeros((1, 1), jnp.float32)
        counts = jnp.sum(cnt_ref[...], axis=1, keepdims=True)
        invc = 1.0 / jnp.maximum(counts, 1.0)
        cls3 = jax.lax.broadcasted_iota(jnp.int32, (_NCPAD, _B), 0)
        w_ref[...] = jnp.sum(jnp.where(lab_ref[...] == cls3, invc, 0.0),
                             axis=0, keepdims=True)

    xb = x_ref[...]
    xn2 = jnp.sum(xb * xb, axis=1, keepdims=True)
    rowinv = 1.0 / jnp.maximum(jnp.sqrt(xn2), 1e-12)
    xs = (xb * rowinv).astype(jnp.bfloat16)
    sim = jax.lax.dot_general(
        xs, ysc_ref[...], (((1,), (1,)), ((), ())),
        preferred_element_type=jnp.float32)

    pos = labrow_ref[...] == lab_ref[...]
    wn = jnp.where(pos, 0.0, sim) * w_ref[...]

    esim = jnp.exp(sim)
    esim_bf = esim.astype(jnp.bfloat16)
    cls_sums = jax.lax.dot_general(
        esim_bf, v_ref[...], (((1,), (0,)), ((), ())),
        preferred_element_type=jnp.float32)
    ucls = jax.lax.broadcasted_iota(jnp.int32, (_BI, _NCPAD), 1)
    uown = (labrow_ref[...] == ucls).astype(jnp.float32)
    pos_sum = jnp.sum(cls_sums * uown, axis=1, keepdims=True)

    neg_inf = jnp.float32(-jnp.inf)
    r = [jnp.full((_BI, _LANES), neg_inf, jnp.float32) for _ in range(_KG)]
    for t in range(_NTILES):
        v = wn[:, t * _LANES:(t + 1) * _LANES]
        for j in range(_KG):
            hi = jnp.maximum(r[j], v)
            v = jnp.minimum(r[j], v)
            r[j] = hi
    planes = r
    thr = None
    for it in range(_K):
        mx = planes[0]
        for j in range(1, _KG):
            mx = jnp.maximum(mx, planes[j])
        mm = jnp.max(mx, axis=1, keepdims=True)
        if it == _K - 1:
            thr = mm
        else:
            planes = [jnp.where(p == mm, neg_inf, p) for p in planes]

    neg_sum = jnp.sum(jnp.where(wn >= thr, esim, 0.0), axis=1, keepdims=True)

    loss = -jnp.log(pos_sum / (pos_sum + neg_sum + 1e-8))
    acc_ref[...] += jnp.sum(loss, axis=0, keepdims=True)

    @pl.when(i == _GRID - 1)
    def _fin():
        out_ref[...] = acc_ref[...] / (jnp.float32(_B) + 1e-8)


def kernel(x, y, labels):
    lab2d = labels.reshape(1, _B).astype(jnp.int32)
    labcol = labels.reshape(_B, 1).astype(jnp.int32)

    counts = _sc_counts(labels.astype(jnp.int32)).reshape(_NCPAD, 16)
    ysc, v = pl.pallas_call(
        _prep_body,
        grid=(1,),
        in_specs=[
            pl.BlockSpec((_B, _D), lambda i: (0, 0)),
            pl.BlockSpec((_B, 1), lambda i: (0, 0)),
        ],
        out_specs=[
            pl.BlockSpec((_B, _D), lambda i: (0, 0)),
            pl.BlockSpec((_B, _NCPAD), lambda i: (0, 0)),
        ],
        out_shape=[
            jax.ShapeDtypeStruct((_B, _D), jnp.bfloat16),
            jax.ShapeDtypeStruct((_B, _NCPAD), jnp.bfloat16),
        ],
    )(y, labcol)

    out = pl.pallas_call(
        _main_body,
        grid=(_GRID,),
        in_specs=[
            pl.BlockSpec((_BI, _D), lambda i: (i, 0)),
            pl.BlockSpec((_B, _D), lambda i: (0, 0)),
            pl.BlockSpec((_NCPAD, 16), lambda i: (0, 0)),
            pl.BlockSpec((1, _B), lambda i: (0, 0)),
            pl.BlockSpec((_BI, 1), lambda i: (i, 0)),
            pl.BlockSpec((_B, _NCPAD), lambda i: (0, 0)),
        ],
        out_specs=pl.BlockSpec((1, 1), lambda i: (0, 0)),
        out_shape=jax.ShapeDtypeStruct((1, 1), jnp.float32),
        scratch_shapes=[pltpu.VMEM((1, 1), jnp.float32),
                        pltpu.VMEM((1, _B), jnp.float32)],
    )(x, ysc, counts, lab2d, labcol, v)
    return out.reshape(())
